# baseline (speedup 1.0000x reference)
import jax
import jax.numpy as jnp
from jax.experimental import pallas as pl
from jax.experimental.pallas import tpu as pltpu

_CHUNKS = 2


def _att_body(x_ref, g_ref, w_ref, u_ref, o_ref, m_ref, l_ref, acc_ref):
    c = pl.program_id(1)
    xb = x_ref[0]
    gb = g_ref[0]
    w = w_ref[...]
    u = u_ref[...]

    t = jnp.tanh(
        jax.lax.dot_general(gb, w, (((1,), (0,)), ((), ())),
                            preferred_element_type=jnp.float32))
    scores = jax.lax.dot_general(u, t, (((1,), (1,)), ((), ())),
                                 preferred_element_type=jnp.float32)
    ones_row = jnp.ones((1, xb.shape[1]), jnp.float32)
    rowsum = jax.lax.dot_general(ones_row, xb, (((1,), (1,)), ((), ())),
                                 preferred_element_type=jnp.float32)

    scores = jnp.where(rowsum == 0.0, jnp.float32(-1e9), scores)
    m_c = jnp.max(scores, axis=-1, keepdims=True)

    @pl.when(c == 0)
    def _init():
        e = jnp.exp(scores - m_c)
        m_ref[...] = m_c
        l_ref[...] = jnp.sum(e, axis=-1, keepdims=True)
        acc_ref[...] = jax.lax.dot_general(
            e, gb, (((1,), (0,)), ((), ())),
            preferred_element_type=jnp.float32)

    @pl.when(c != 0)
    def _update():
        m_old = m_ref[...]
        m_new = jnp.maximum(m_old, m_c)
        corr = jnp.exp(m_old - m_new)
        e = jnp.exp(scores - m_new)
        m_ref[...] = m_new
        l_ref[...] = l_ref[...] * corr + jnp.sum(e, axis=-1, keepdims=True)
        acc_ref[...] = acc_ref[...] * corr + jax.lax.dot_general(
            e, gb, (((1,), (0,)), ((), ())),
            preferred_element_type=jnp.float32)

    @pl.when(c == _CHUNKS - 1)
    def _finalize():
        o_ref[0] = acc_ref[...] / l_ref[...]


def kernel(x, gru_output, w_omega, u_omega):
    B, S, D = x.shape
    A = w_omega.shape[1]
    u2 = u_omega.reshape(1, A)
    sc = S // _CHUNKS
    return pl.pallas_call(
        _att_body,
        grid=(B, _CHUNKS),
        in_specs=[
            pl.BlockSpec((1, sc, D), lambda b, c: (b, c, 0)),
            pl.BlockSpec((1, sc, D), lambda b, c: (b, c, 0)),
            pl.BlockSpec((D, A), lambda b, c: (0, 0)),
            pl.BlockSpec((1, A), lambda b, c: (0, 0)),
        ],
        out_specs=pl.BlockSpec((1, 1, D), lambda b, c: (b, 0, 0)),
        out_shape=jax.ShapeDtypeStruct((B, 1, D), jnp.float32),
        scratch_shapes=[
            pltpu.VMEM((1, 1), jnp.float32),
            pltpu.VMEM((1, 1), jnp.float32),
            pltpu.VMEM((1, D), jnp.float32),
        ],
        compiler_params=pltpu.CompilerParams(
            dimension_semantics=("parallel", "arbitrary"),
            vmem_limit_bytes=56 * 1024 * 1024,
        ),
        name="fused_additive_attention",
    )(x, gru_output, w_omega, u2).reshape(B, D)
